# Initial kernel scaffold; baseline (speedup 1.0000x reference)
import jax
import jax.numpy as jnp
from jax.experimental import pallas as pl


def kernel(x, edge_index, Wf1, bf1, Wf2, bf2, Ws1, bs1, Ws2, bs2, Wc1, bc1, Wc2, bc2):
    raise NotImplementedError("write your pallas kernel here")


# speedup vs baseline: 15.4027x; 15.4027x over previous
import functools

import jax
import jax.numpy as jnp
from jax import lax
from jax.experimental import pallas as pl
from jax.experimental.pallas import tpu as pltpu
from jax.experimental.pallas import tpu_sc as plsc

N_NODES = 10000
N_PAD = 10240
E_EDGES = 320000
D_FEAT = 128

NC = 2
NS = 16
NW = NC * NS
E_PER_W = E_EDGES // NW
CHUNK = 80
N_CHUNKS = E_PER_W // CHUNK
ROWS_PER_TILE = N_PAD // NS


def _sc_aggregate(row3, col3, x):
  mesh = plsc.VectorSubcoreMesh(
      core_axis_name="c", subcore_axis_name="s", num_cores=NC, num_subcores=NS)

  @functools.partial(
      pl.kernel,
      mesh=mesh,
      out_type=[
          jax.ShapeDtypeStruct((NC, N_PAD, D_FEAT), jnp.float32),
          jax.ShapeDtypeStruct((NC, N_PAD), jnp.float32),
      ],
      scratch_types=[
          pltpu.VMEM((N_CHUNKS, CHUNK), jnp.int32),
          pltpu.VMEM((N_CHUNKS, CHUNK), jnp.int32),
          pltpu.VMEM((CHUNK, D_FEAT), jnp.float32),
          pltpu.VMEM((CHUNK,), jnp.float32),
          pltpu.VMEM_SHARED((N_PAD, D_FEAT), jnp.float32),
          pltpu.VMEM_SHARED((N_PAD,), jnp.float32),
          pltpu.SemaphoreType.DMA,
      ],
  )
  def agg(row_hbm, col_hbm, x_hbm, part_out, deg_out,
          row_v, col_v, rows_v, ones_v, acc_sh, deg_sh, sem):
    cid = lax.axis_index("c")
    sid = lax.axis_index("s")
    wid = sid * NC + cid

    zeros16 = jnp.zeros((16,), jnp.float32)

    def zrow(i, carry):
      for j in range(D_FEAT // 16):
        rows_v[i, pl.ds(j * 16, 16)] = zeros16
      return carry
    lax.fori_loop(0, CHUNK, zrow, 0)
    for j in range(CHUNK // 16):
      ones_v[pl.ds(j * 16, 16)] = jnp.ones((16,), jnp.float32)

    base = sid * ROWS_PER_TILE
    for k in range(ROWS_PER_TILE // CHUNK):
      pltpu.sync_copy(rows_v, acc_sh.at[pl.ds(base + k * CHUNK, CHUNK)])
    for k in range(ROWS_PER_TILE // D_FEAT):
      pltpu.sync_copy(rows_v.at[0], deg_sh.at[pl.ds(base + k * D_FEAT, D_FEAT)])

    pltpu.sync_copy(row_hbm.at[wid], row_v)
    pltpu.sync_copy(col_hbm.at[wid], col_v)

    plsc.subcore_barrier()

    def step(j, carry):
      pltpu.async_copy(x_hbm.at[row_v.at[j]], rows_v, sem).wait()
      pltpu.sync_copy(rows_v, acc_sh.at[col_v.at[j]], add=True)
      pltpu.sync_copy(ones_v, deg_sh.at[col_v.at[j]], add=True)
      return carry
    lax.fori_loop(0, N_CHUNKS, step, 0)

    plsc.subcore_barrier()

    pltpu.sync_copy(acc_sh.at[pl.ds(base, ROWS_PER_TILE)],
                    part_out.at[cid].at[pl.ds(base, ROWS_PER_TILE)])
    pltpu.sync_copy(deg_sh.at[pl.ds(base, ROWS_PER_TILE)],
                    deg_out.at[cid].at[pl.ds(base, ROWS_PER_TILE)])

  return agg(row3, col3, x)


def _tc_mlps(x, part, degp, Wf1, bf1, Wf2, bf2, Ws1, bs1, Ws2, bs2,
             Wc1a, Wc1b, bc1, Wc2, bc2):
  BLK = 400
  grid = (N_NODES // BLK,)

  def body(x_ref, p_ref, d_ref, wf1, bf1r, wf2, bf2r, ws1, bs1r, ws2, bs2r,
           wc1a, wc1b, bc1r, wc2, bc2r, o_ref):
    xb = x_ref[...]
    hf = jnp.maximum(jnp.dot(xb, wf1[...],
                             preferred_element_type=jnp.float32) + bf1r[...],
                     0.0)
    hf = jnp.dot(hf, wf2[...], preferred_element_type=jnp.float32) + bf2r[...]

    p = p_ref[0] + p_ref[1]
    d = d_ref[0, :, :] + d_ref[1, :, :]
    ax = p * (1.0 / jnp.maximum(d, 1.0))
    hs = jnp.maximum(jnp.dot(ax, ws1[...],
                             preferred_element_type=jnp.float32) + bs1r[...],
                     0.0)
    hs = jnp.dot(hs, ws2[...], preferred_element_type=jnp.float32) + bs2r[...]

    h1 = jnp.maximum(jnp.dot(hf, wc1a[...], preferred_element_type=jnp.float32)
                     + jnp.dot(hs, wc1b[...],
                               preferred_element_type=jnp.float32)
                     + bc1r[...], 0.0)
    o_ref[...] = (jnp.dot(h1, wc2[...], preferred_element_type=jnp.float32)
                  + bc2r[...])

  full = lambda shape: pl.BlockSpec(shape, lambda i: (0,) * len(shape))
  return pl.pallas_call(
      body,
      grid=grid,
      in_specs=[
          pl.BlockSpec((BLK, D_FEAT), lambda i: (i, 0)),
          pl.BlockSpec((NC, BLK, D_FEAT), lambda i: (0, i, 0)),
          pl.BlockSpec((NC, BLK, 1), lambda i: (0, i, 0)),
          full((D_FEAT, 128)), full((1, 128)),
          full((128, 128)), full((1, 128)),
          full((D_FEAT, 128)), full((1, 128)),
          full((128, 128)), full((1, 128)),
          full((128, 128)), full((128, 128)), full((1, 128)),
          full((128, 64)), full((1, 64)),
      ],
      out_specs=pl.BlockSpec((BLK, 64), lambda i: (i, 0)),
      out_shape=jax.ShapeDtypeStruct((N_NODES, 64), jnp.float32),
  )(x, part, degp, Wf1, bf1, Wf2, bf2, Ws1, bs1, Ws2, bs2,
    Wc1a, Wc1b, bc1, Wc2, bc2)


def kernel(x, edge_index, Wf1, bf1, Wf2, bf2, Ws1, bs1, Ws2, bs2,
           Wc1, bc1, Wc2, bc2):
  row3 = edge_index[0].reshape(NW, N_CHUNKS, CHUNK)
  col3 = edge_index[1].reshape(NW, N_CHUNKS, CHUNK)

  part, degp = _sc_aggregate(row3, col3, x)

  out = _tc_mlps(
      x, part, degp.reshape(NC, N_PAD, 1),
      Wf1, bf1.reshape(1, 128), Wf2, bf2.reshape(1, 128),
      Ws1, bs1.reshape(1, 128), Ws2, bs2.reshape(1, 128),
      Wc1[:128], Wc1[128:], bc1.reshape(1, 128),
      Wc2, bc2.reshape(1, 64))
  return out
